# baseline (speedup 1.0000x reference)
import jax
import jax.numpy as jnp
from jax.experimental import pallas as pl
from jax.experimental.pallas import tpu as pltpu

OUT_H = 7
OUT_W = 7
WIN = 6
K = 8


def _roi_pool_body(params_ref, f_ref, out_ref, tmp_ref):
    neg = jnp.float32(-jnp.inf)
    for k in range(K):
        b = params_ref[k, 0]
        cols = []
        for wb in range(OUT_W):
            s = params_ref[k, 1 + wb]
            e = params_ref[k, 8 + wb]
            s0 = jnp.minimum(s, 32 - WIN)
            win = f_ref[b, pl.ds(s0, WIN)]
            idx = s0 + jax.lax.broadcasted_iota(jnp.int32, (WIN, 1, 1), 0)
            m = (idx >= s) & (idx < e)
            cols.append(jnp.max(jnp.where(m, win, neg), axis=0))
        tmp_ref[...] = jnp.stack(cols, axis=1)
        rows = []
        for hb in range(OUT_H):
            s = params_ref[k, 15 + hb]
            e = params_ref[k, 22 + hb]
            s0 = jnp.minimum(s, 32 - WIN)
            win = tmp_ref[pl.ds(s0, WIN)]
            idx = s0 + jax.lax.broadcasted_iota(jnp.int32, (WIN, 1, 1), 0)
            m = (idx >= s) & (idx < e)
            rows.append(jnp.max(jnp.where(m, win, neg), axis=0))
        pooled = jnp.stack(rows, axis=0)
        pooled = jnp.where(pooled > neg, pooled, jnp.float32(0.0))
        out_ref[k] = pooled.reshape(OUT_H * OUT_W, 96)


def _bin_params(rois):
    rois_i = rois.astype(jnp.int32)
    start_w = rois_i[:, 1].astype(jnp.float32)
    start_h = rois_i[:, 2].astype(jnp.float32)
    end_w = rois_i[:, 3].astype(jnp.float32)
    end_h = rois_i[:, 4].astype(jnp.float32)
    bin_h = jnp.maximum(end_h - start_h, 1.0) / float(OUT_H)
    bin_w = jnp.maximum(end_w - start_w, 1.0) / float(OUT_W)
    hs = jnp.arange(OUT_H, dtype=jnp.float32)
    ws = jnp.arange(OUT_W, dtype=jnp.float32)
    h_start = jnp.floor(hs[None, :] * bin_h[:, None] + start_h[:, None])
    h_end = jnp.ceil((hs[None, :] + 1.0) * bin_h[:, None] + start_h[:, None])
    w_start = jnp.floor(ws[None, :] * bin_w[:, None] + start_w[:, None])
    w_end = jnp.ceil((ws[None, :] + 1.0) * bin_w[:, None] + start_w[:, None])
    clip = lambda a: jnp.clip(a, 0, 32).astype(jnp.int32)
    return jnp.concatenate(
        [
            rois_i[:, :1],
            clip(w_start), clip(w_end), clip(h_start), clip(h_end),
            jnp.zeros((rois_i.shape[0], 3), jnp.int32),
        ],
        axis=1,
    )


def kernel(features, rois):
    N = rois.shape[0]
    C = features.shape[1]
    params = _bin_params(rois)
    fT = jnp.transpose(features, (0, 3, 2, 1))
    out = pl.pallas_call(
        _roi_pool_body,
        grid=(N // K,),
        in_specs=[
            pl.BlockSpec((K, 32), lambda i: (i, 0), memory_space=pltpu.SMEM),
            pl.BlockSpec((32, 32, 32, C), lambda i: (0, 0, 0, 0)),
        ],
        out_specs=pl.BlockSpec((K, OUT_H * OUT_W, C), lambda i: (i, 0, 0)),
        out_shape=jax.ShapeDtypeStruct((N, OUT_H * OUT_W, C), jnp.float32),
        scratch_shapes=[pltpu.VMEM((32, OUT_W, C), jnp.float32)],
        compiler_params=pltpu.CompilerParams(
            dimension_semantics=("parallel",),
        ),
    )(params, fT)
    return jnp.transpose(out.reshape(N, OUT_H, OUT_W, C), (0, 3, 1, 2))


# speedup vs baseline: 1.4892x; 1.4892x over previous
import jax
import jax.numpy as jnp
from jax.experimental import pallas as pl
from jax.experimental.pallas import tpu as pltpu

OUT_H = 7
OUT_W = 7
WIN = 6
K = 8


def _roi_pool_body(params_ref, f_ref, out_ref, p_ref, tmp_ref):
    neg = jnp.float32(-jnp.inf)

    @pl.when(pl.program_id(0) == 0)
    def _build_pyramid():
        p_ref[0, :, 0:31] = jnp.maximum(f_ref[:, 0:31], f_ref[:, 1:32])
        p_ref[0, :, 31] = f_ref[:, 31]
        p_ref[1, :, 0:29] = jnp.maximum(p_ref[0, :, 0:29], p_ref[0, :, 2:31])

    for k in range(K):
        b = params_ref[k, 0]
        for wb in range(OUT_W):
            kw = params_ref[k, 1 + wb]
            o1 = params_ref[k, 8 + wb]
            o2 = params_ref[k, 15 + wb]
            v = params_ref[k, 22 + wb]
            kc = jnp.maximum(kw - 1, 0)
            col0 = f_ref[b, o1]
            colp = jnp.maximum(p_ref[kc, b, o1], p_ref[kc, b, o2])
            col = jnp.where(kw > 0, colp, col0)
            tmp_ref[wb] = jnp.where(v > 0, col, neg)
        for hb in range(OUT_H):
            s = params_ref[k, 29 + hb]
            e = params_ref[k, 36 + hb]
            s0 = jnp.minimum(s, 32 - WIN)
            win = tmp_ref[:, pl.ds(s0, WIN), :]
            idx = s0 + jax.lax.broadcasted_iota(jnp.int32, (1, WIN, 1), 1)
            m = (idx >= s) & (idx < e)
            row = jnp.max(jnp.where(m, win, neg), axis=1)
            out_ref[k, hb] = jnp.where(row > neg, row, jnp.float32(0.0))


def _bin_params(rois):
    rois_i = rois.astype(jnp.int32)
    start_w = rois_i[:, 1].astype(jnp.float32)
    start_h = rois_i[:, 2].astype(jnp.float32)
    end_w = rois_i[:, 3].astype(jnp.float32)
    end_h = rois_i[:, 4].astype(jnp.float32)
    bin_h = jnp.maximum(end_h - start_h, 1.0) / float(OUT_H)
    bin_w = jnp.maximum(end_w - start_w, 1.0) / float(OUT_W)
    hs = jnp.arange(OUT_H, dtype=jnp.float32)
    ws = jnp.arange(OUT_W, dtype=jnp.float32)
    clip = lambda a: jnp.clip(a, 0, 32).astype(jnp.int32)
    h_start = clip(jnp.floor(hs[None, :] * bin_h[:, None] + start_h[:, None]))
    h_end = clip(jnp.ceil((hs[None, :] + 1.0) * bin_h[:, None] + start_h[:, None]))
    w_start = clip(jnp.floor(ws[None, :] * bin_w[:, None] + start_w[:, None]))
    w_end = clip(jnp.ceil((ws[None, :] + 1.0) * bin_w[:, None] + start_w[:, None]))
    wlen = w_end - w_start
    kw = jnp.where(wlen >= 4, 2, jnp.where(wlen >= 2, 1, 0))
    pw = jnp.int32(1) << kw
    o1 = jnp.clip(w_start, 0, 32 - pw)
    o2 = jnp.clip(w_end - pw, 0, 32 - pw)
    valid = (wlen > 0).astype(jnp.int32)
    return jnp.concatenate(
        [
            rois_i[:, :1],
            kw, o1, o2, valid, h_start, h_end,
            jnp.zeros((rois_i.shape[0], 21), jnp.int32),
        ],
        axis=1,
    )


def kernel(features, rois):
    N = rois.shape[0]
    C = features.shape[1]
    params = _bin_params(rois)
    fT = jnp.transpose(features, (0, 3, 2, 1))
    out = pl.pallas_call(
        _roi_pool_body,
        grid=(N // K,),
        in_specs=[
            pl.BlockSpec((K, 64), lambda i: (i, 0), memory_space=pltpu.SMEM),
            pl.BlockSpec((32, 32, 32, C), lambda i: (0, 0, 0, 0)),
        ],
        out_specs=pl.BlockSpec((K, OUT_H, OUT_W, C), lambda i: (i, 0, 0, 0)),
        out_shape=jax.ShapeDtypeStruct((N, OUT_H, OUT_W, C), jnp.float32),
        scratch_shapes=[
            pltpu.VMEM((2, 32, 32, 32, C), jnp.float32),
            pltpu.VMEM((OUT_W, 32, C), jnp.float32),
        ],
        compiler_params=pltpu.CompilerParams(
            dimension_semantics=("arbitrary",),
        ),
    )(params, fT)
    return jnp.transpose(out, (0, 3, 1, 2))


# speedup vs baseline: 1.5938x; 1.0702x over previous
import jax
import jax.numpy as jnp
from jax.experimental import pallas as pl
from jax.experimental.pallas import tpu as pltpu

OUT_H = 7
OUT_W = 7
WIN = 6
K = 8


def _roi_pool_body(params_ref, f_ref, t_ref, out_ref, p_ref, tmp_ref):
    neg = jnp.float32(-jnp.inf)

    @pl.when(pl.program_id(0) == 0)
    def _build_pyramid():
        p_ref[0, :, 0:31] = jnp.maximum(f_ref[:, 0:31], f_ref[:, 1:32])
        p_ref[0, :, 31] = f_ref[:, 31]
        p_ref[1, :, 0:29] = jnp.maximum(p_ref[0, :, 0:29], p_ref[0, :, 2:31])

    for k in range(K):
        b = params_ref[k, 0]
        for wb in range(OUT_W):
            kw = params_ref[k, 1 + wb]
            o1 = params_ref[k, 8 + wb]
            o2 = params_ref[k, 15 + wb]
            v = params_ref[k, 22 + wb]
            kc = jnp.maximum(kw - 1, 0)
            col0 = f_ref[b, o1]
            colp = jnp.maximum(p_ref[kc, b, o1], p_ref[kc, b, o2])
            col = jnp.where(kw > 0, colp, col0)
            tmp_ref[wb] = jnp.where(v > 0, col, neg)
        for hb in range(OUT_H):
            s0 = params_ref[k, 29 + hb]
            mi = params_ref[k, 36 + hb]
            win = tmp_ref[:, pl.ds(s0, WIN), :]
            row = jnp.max(win + t_ref[mi][None], axis=1)
            out_ref[k, hb] = jnp.where(row > neg, row, jnp.float32(0.0))


def _bin_params(rois):
    rois_i = rois.astype(jnp.int32)
    start_w = rois_i[:, 1].astype(jnp.float32)
    start_h = rois_i[:, 2].astype(jnp.float32)
    end_w = rois_i[:, 3].astype(jnp.float32)
    end_h = rois_i[:, 4].astype(jnp.float32)
    bin_h = jnp.maximum(end_h - start_h, 1.0) / float(OUT_H)
    bin_w = jnp.maximum(end_w - start_w, 1.0) / float(OUT_W)
    hs = jnp.arange(OUT_H, dtype=jnp.float32)
    ws = jnp.arange(OUT_W, dtype=jnp.float32)
    clip = lambda a: jnp.clip(a, 0, 32).astype(jnp.int32)
    h_start = clip(jnp.floor(hs[None, :] * bin_h[:, None] + start_h[:, None]))
    h_end = clip(jnp.ceil((hs[None, :] + 1.0) * bin_h[:, None] + start_h[:, None]))
    w_start = clip(jnp.floor(ws[None, :] * bin_w[:, None] + start_w[:, None]))
    w_end = clip(jnp.ceil((ws[None, :] + 1.0) * bin_w[:, None] + start_w[:, None]))
    wlen = w_end - w_start
    kw = jnp.where(wlen >= 4, 2, jnp.where(wlen >= 2, 1, 0))
    pw = jnp.int32(1) << kw
    o1 = jnp.clip(w_start, 0, 32 - pw)
    o2 = jnp.clip(w_end - pw, 0, 32 - pw)
    valid = (wlen > 0).astype(jnp.int32)
    hs0 = jnp.clip(h_start, 0, 32 - WIN)
    mi = (h_start - hs0) * 7 + (h_end - hs0)
    return jnp.concatenate(
        [
            rois_i[:, :1],
            kw, o1, o2, valid, hs0, mi,
            jnp.zeros((rois_i.shape[0], 21), jnp.int32),
        ],
        axis=1,
    )


def kernel(features, rois):
    N = rois.shape[0]
    C = features.shape[1]
    params = _bin_params(rois)
    fT = jnp.transpose(features, (0, 3, 2, 1))
    d = jnp.arange(WIN, dtype=jnp.int32)
    off = jnp.arange(49, dtype=jnp.int32) // 7
    end = jnp.arange(49, dtype=jnp.int32) % 7
    tbl = jnp.where(
        (d[None, :] >= off[:, None]) & (d[None, :] < end[:, None]),
        jnp.float32(0.0), jnp.float32(-jnp.inf),
    )
    tbl = jnp.broadcast_to(tbl[:, :, None], (49, WIN, C))
    out = pl.pallas_call(
        _roi_pool_body,
        grid=(N // K,),
        in_specs=[
            pl.BlockSpec((K, 64), lambda i: (i, 0), memory_space=pltpu.SMEM),
            pl.BlockSpec((32, 32, 32, C), lambda i: (0, 0, 0, 0)),
            pl.BlockSpec((49, WIN, C), lambda i: (0, 0, 0)),
        ],
        out_specs=pl.BlockSpec((K, OUT_H, OUT_W, C), lambda i: (i, 0, 0, 0)),
        out_shape=jax.ShapeDtypeStruct((N, OUT_H, OUT_W, C), jnp.float32),
        scratch_shapes=[
            pltpu.VMEM((2, 32, 32, 32, C), jnp.float32),
            pltpu.VMEM((OUT_W, 32, C), jnp.float32),
        ],
        compiler_params=pltpu.CompilerParams(
            dimension_semantics=("arbitrary",),
        ),
    )(params, fT, tbl)
    return jnp.transpose(out, (0, 3, 1, 2))


# speedup vs baseline: 1.6796x; 1.0539x over previous
import jax
import jax.numpy as jnp
from jax.experimental import pallas as pl
from jax.experimental.pallas import tpu as pltpu

OUT_H = 7
OUT_W = 7
WIN = 6
K = 8


def _roi_pool_body(params_ref, f_ref, t_ref, out_ref, p_ref, tmp_ref, sem):
    neg = jnp.float32(-jnp.inf)

    @pl.when(pl.program_id(0) == 0)
    def _build_pyramid():
        cp = pltpu.make_async_copy(f_ref, p_ref.at[0], sem)
        cp.start()
        cp.wait()
        p_ref[1, :, 0:31] = jnp.maximum(p_ref[0, :, 0:31], p_ref[0, :, 1:32])
        p_ref[1, :, 31] = p_ref[0, :, 31]
        p_ref[2, :, 0:29] = jnp.maximum(p_ref[1, :, 0:29], p_ref[1, :, 2:31])

    for k in range(K):
        b = params_ref[k, 0]
        for wb in range(OUT_W):
            kw = params_ref[k, 1 + wb]
            o1 = params_ref[k, 8 + wb]
            o2 = params_ref[k, 15 + wb]
            v = params_ref[k, 22 + wb]
            col = jnp.maximum(p_ref[kw, b, o1], p_ref[kw, b, o2])
            tmp_ref[k, wb] = jnp.where(v > 0, col, neg)
    for k in range(K):
        for hb in range(OUT_H):
            s0 = params_ref[k, 29 + hb]
            mi = params_ref[k, 36 + hb]
            win = tmp_ref[k, :, pl.ds(s0, WIN), :]
            row = jnp.max(win + t_ref[mi][None], axis=1)
            out_ref[k, hb] = jnp.where(row > neg, row, jnp.float32(0.0))


def _bin_params(rois):
    rois_i = rois.astype(jnp.int32)
    start_w = rois_i[:, 1].astype(jnp.float32)
    start_h = rois_i[:, 2].astype(jnp.float32)
    end_w = rois_i[:, 3].astype(jnp.float32)
    end_h = rois_i[:, 4].astype(jnp.float32)
    bin_h = jnp.maximum(end_h - start_h, 1.0) / float(OUT_H)
    bin_w = jnp.maximum(end_w - start_w, 1.0) / float(OUT_W)
    hs = jnp.arange(OUT_H, dtype=jnp.float32)
    ws = jnp.arange(OUT_W, dtype=jnp.float32)
    clip = lambda a: jnp.clip(a, 0, 32).astype(jnp.int32)
    h_start = clip(jnp.floor(hs[None, :] * bin_h[:, None] + start_h[:, None]))
    h_end = clip(jnp.ceil((hs[None, :] + 1.0) * bin_h[:, None] + start_h[:, None]))
    w_start = clip(jnp.floor(ws[None, :] * bin_w[:, None] + start_w[:, None]))
    w_end = clip(jnp.ceil((ws[None, :] + 1.0) * bin_w[:, None] + start_w[:, None]))
    wlen = w_end - w_start
    kw = jnp.where(wlen >= 4, 2, jnp.where(wlen >= 2, 1, 0))
    pw = jnp.int32(1) << kw
    o1 = jnp.clip(w_start, 0, 32 - pw)
    o2 = jnp.clip(w_end - pw, 0, 32 - pw)
    valid = (wlen > 0).astype(jnp.int32)
    hs0 = jnp.clip(h_start, 0, 32 - WIN)
    mi = (h_start - hs0) * 7 + (h_end - hs0)
    return jnp.concatenate(
        [
            rois_i[:, :1],
            kw, o1, o2, valid, hs0, mi,
            jnp.zeros((rois_i.shape[0], 21), jnp.int32),
        ],
        axis=1,
    )


def kernel(features, rois):
    N = rois.shape[0]
    C = features.shape[1]
    params = _bin_params(rois)
    fT = jnp.transpose(features, (0, 3, 2, 1))
    d = jnp.arange(WIN, dtype=jnp.int32)
    off = jnp.arange(49, dtype=jnp.int32) // 7
    end = jnp.arange(49, dtype=jnp.int32) % 7
    tbl = jnp.where(
        (d[None, :] >= off[:, None]) & (d[None, :] < end[:, None]),
        jnp.float32(0.0), jnp.float32(-jnp.inf),
    )
    tbl = jnp.broadcast_to(tbl[:, :, None], (49, WIN, C))
    out = pl.pallas_call(
        _roi_pool_body,
        grid=(N // K,),
        in_specs=[
            pl.BlockSpec((K, 64), lambda i: (i, 0), memory_space=pltpu.SMEM),
            pl.BlockSpec(memory_space=pltpu.MemorySpace.HBM),
            pl.BlockSpec((49, WIN, C), lambda i: (0, 0, 0)),
        ],
        out_specs=pl.BlockSpec((K, OUT_H, OUT_W, C), lambda i: (i, 0, 0, 0)),
        out_shape=jax.ShapeDtypeStruct((N, OUT_H, OUT_W, C), jnp.float32),
        scratch_shapes=[
            pltpu.VMEM((3, 32, 32, 32, C), jnp.float32),
            pltpu.VMEM((K, OUT_W, 32, C), jnp.float32),
            pltpu.SemaphoreType.DMA,
        ],
        compiler_params=pltpu.CompilerParams(
            dimension_semantics=("arbitrary",),
        ),
    )(params, fT, tbl)
    return jnp.transpose(out, (0, 3, 1, 2))


# speedup vs baseline: 1.7462x; 1.0396x over previous
import jax
import jax.numpy as jnp
from jax.experimental import pallas as pl
from jax.experimental.pallas import tpu as pltpu

OUT_H = 7
OUT_W = 7
WIN = 6
K = 8


def _roi_pool_body(params_ref, f_ref, t_ref, out_ref, p_ref, tmp_ref, sem):
    neg = jnp.float32(-jnp.inf)

    @pl.when(pl.program_id(0) == 0)
    def _build_pyramid():
        cp = pltpu.make_async_copy(f_ref, p_ref.at[0, :, 0:32], sem)
        cp.start()
        cp.wait()
        p_ref[1, :, 0:31] = jnp.maximum(p_ref[0, :, 0:31], p_ref[0, :, 1:32])
        p_ref[1, :, 31] = p_ref[0, :, 31]
        p_ref[2, :, 0:29] = jnp.maximum(p_ref[1, :, 0:29], p_ref[1, :, 2:31])
        p_ref[:, :, 32] = jnp.full((3, 32, 32, 96), neg)

    for k in range(K):
        b = params_ref[k, 0]
        for wb in range(OUT_W):
            kw = params_ref[k, 1 + wb]
            o1 = params_ref[k, 8 + wb]
            o2 = params_ref[k, 15 + wb]
            tmp_ref[k, wb] = jnp.maximum(p_ref[kw, b, o1], p_ref[kw, b, o2])
    for k in range(K):
        for hb in range(OUT_H):
            s0 = params_ref[k, 29 + hb]
            mi = params_ref[k, 36 + hb]
            win = tmp_ref[k, :, pl.ds(s0, WIN), :]
            row = jnp.max(win + t_ref[mi][None], axis=1)
            out_ref[k, hb] = jnp.where(row > neg, row, jnp.float32(0.0))


def _bin_params(rois):
    rois_i = rois.astype(jnp.int32)
    start_w = rois_i[:, 1].astype(jnp.float32)
    start_h = rois_i[:, 2].astype(jnp.float32)
    end_w = rois_i[:, 3].astype(jnp.float32)
    end_h = rois_i[:, 4].astype(jnp.float32)
    bin_h = jnp.maximum(end_h - start_h, 1.0) / float(OUT_H)
    bin_w = jnp.maximum(end_w - start_w, 1.0) / float(OUT_W)
    hs = jnp.arange(OUT_H, dtype=jnp.float32)
    ws = jnp.arange(OUT_W, dtype=jnp.float32)
    clip = lambda a: jnp.clip(a, 0, 32).astype(jnp.int32)
    h_start = clip(jnp.floor(hs[None, :] * bin_h[:, None] + start_h[:, None]))
    h_end = clip(jnp.ceil((hs[None, :] + 1.0) * bin_h[:, None] + start_h[:, None]))
    w_start = clip(jnp.floor(ws[None, :] * bin_w[:, None] + start_w[:, None]))
    w_end = clip(jnp.ceil((ws[None, :] + 1.0) * bin_w[:, None] + start_w[:, None]))
    wlen = w_end - w_start
    kw = jnp.where(wlen >= 4, 2, jnp.where(wlen >= 2, 1, 0))
    pw = jnp.int32(1) << kw
    o1 = jnp.clip(w_start, 0, 32 - pw)
    o2 = jnp.clip(w_end - pw, 0, 32 - pw)
    o1 = jnp.where(wlen > 0, o1, 32)
    o2 = jnp.where(wlen > 0, o2, 32)
    valid = (wlen > 0).astype(jnp.int32)
    hs0 = jnp.clip(h_start, 0, 32 - WIN)
    mi = (h_start - hs0) * 7 + (h_end - hs0)
    return jnp.concatenate(
        [
            rois_i[:, :1],
            kw, o1, o2, valid, hs0, mi,
            jnp.zeros((rois_i.shape[0], 21), jnp.int32),
        ],
        axis=1,
    )


def kernel(features, rois):
    N = rois.shape[0]
    C = features.shape[1]
    params = _bin_params(rois)
    fT = jnp.transpose(features, (0, 3, 2, 1))
    d = jnp.arange(WIN, dtype=jnp.int32)
    off = jnp.arange(49, dtype=jnp.int32) // 7
    end = jnp.arange(49, dtype=jnp.int32) % 7
    tbl = jnp.where(
        (d[None, :] >= off[:, None]) & (d[None, :] < end[:, None]),
        jnp.float32(0.0), jnp.float32(-jnp.inf),
    )
    tbl = jnp.broadcast_to(tbl[:, :, None], (49, WIN, C))
    out = pl.pallas_call(
        _roi_pool_body,
        grid=(N // K,),
        in_specs=[
            pl.BlockSpec((K, 64), lambda i: (i, 0), memory_space=pltpu.SMEM),
            pl.BlockSpec(memory_space=pltpu.MemorySpace.HBM),
            pl.BlockSpec((49, WIN, C), lambda i: (0, 0, 0)),
        ],
        out_specs=pl.BlockSpec((K, OUT_H, OUT_W, C), lambda i: (i, 0, 0, 0)),
        out_shape=jax.ShapeDtypeStruct((N, OUT_H, OUT_W, C), jnp.float32),
        scratch_shapes=[
            pltpu.VMEM((3, 32, 33, 32, C), jnp.float32),
            pltpu.VMEM((K, OUT_W, 32, C), jnp.float32),
            pltpu.SemaphoreType.DMA,
        ],
        compiler_params=pltpu.CompilerParams(
            dimension_semantics=("arbitrary",),
        ),
    )(params, fT, tbl)
    return jnp.transpose(out, (0, 3, 1, 2))


# speedup vs baseline: 1.7968x; 1.0290x over previous
import jax
import jax.numpy as jnp
from jax.experimental import pallas as pl
from jax.experimental.pallas import tpu as pltpu

OUT_H = 7
OUT_W = 7
WIN = 6
K = 40


def _roi_pool_body(params_ref, f_ref, t_ref, out_ref, p_ref, tmp_ref, sem):
    neg = jnp.float32(-jnp.inf)

    @pl.when(pl.program_id(0) == 0)
    def _build_pyramid():
        cp = pltpu.make_async_copy(f_ref, p_ref.at[0, :, 0:32], sem)
        cp.start()
        cp.wait()
        p_ref[1, :, 0:31] = jnp.maximum(p_ref[0, :, 0:31], p_ref[0, :, 1:32])
        p_ref[1, :, 31] = p_ref[0, :, 31]
        p_ref[2, :, 0:29] = jnp.maximum(p_ref[1, :, 0:29], p_ref[1, :, 2:31])
        p_ref[:, :, 32] = jnp.full((3, 32, 32, 96), neg)

    for k in range(K):
        b = params_ref[k, 0]
        for wb in range(OUT_W):
            kw = params_ref[k, 1 + wb]
            o1 = params_ref[k, 8 + wb]
            o2 = params_ref[k, 15 + wb]
            tmp_ref[k, wb] = jnp.maximum(p_ref[kw, b, o1], p_ref[kw, b, o2])
    for k in range(K):
        for hb in range(OUT_H):
            s0 = params_ref[k, 29 + hb]
            mi = params_ref[k, 36 + hb]
            win = tmp_ref[k, :, pl.ds(s0, WIN), :]
            row = jnp.max(win + t_ref[mi][None], axis=1)
            out_ref[k, hb] = jnp.where(row > neg, row, jnp.float32(0.0))


def _bin_params(rois):
    rois_i = rois.astype(jnp.int32)
    start_w = rois_i[:, 1].astype(jnp.float32)
    start_h = rois_i[:, 2].astype(jnp.float32)
    end_w = rois_i[:, 3].astype(jnp.float32)
    end_h = rois_i[:, 4].astype(jnp.float32)
    bin_h = jnp.maximum(end_h - start_h, 1.0) / float(OUT_H)
    bin_w = jnp.maximum(end_w - start_w, 1.0) / float(OUT_W)
    hs = jnp.arange(OUT_H, dtype=jnp.float32)
    ws = jnp.arange(OUT_W, dtype=jnp.float32)
    clip = lambda a: jnp.clip(a, 0, 32).astype(jnp.int32)
    h_start = clip(jnp.floor(hs[None, :] * bin_h[:, None] + start_h[:, None]))
    h_end = clip(jnp.ceil((hs[None, :] + 1.0) * bin_h[:, None] + start_h[:, None]))
    w_start = clip(jnp.floor(ws[None, :] * bin_w[:, None] + start_w[:, None]))
    w_end = clip(jnp.ceil((ws[None, :] + 1.0) * bin_w[:, None] + start_w[:, None]))
    wlen = w_end - w_start
    kw = jnp.where(wlen >= 4, 2, jnp.where(wlen >= 2, 1, 0))
    pw = jnp.int32(1) << kw
    o1 = jnp.clip(w_start, 0, 32 - pw)
    o2 = jnp.clip(w_end - pw, 0, 32 - pw)
    o1 = jnp.where(wlen > 0, o1, 32)
    o2 = jnp.where(wlen > 0, o2, 32)
    valid = (wlen > 0).astype(jnp.int32)
    hs0 = jnp.clip(h_start, 0, 32 - WIN)
    mi = (h_start - hs0) * 7 + (h_end - hs0)
    return jnp.concatenate(
        [
            rois_i[:, :1],
            kw, o1, o2, valid, hs0, mi,
            jnp.zeros((rois_i.shape[0], 21), jnp.int32),
        ],
        axis=1,
    )


def kernel(features, rois):
    N = rois.shape[0]
    C = features.shape[1]
    params = _bin_params(rois)
    fT = jnp.transpose(features, (0, 3, 2, 1))
    d = jnp.arange(WIN, dtype=jnp.int32)
    off = jnp.arange(49, dtype=jnp.int32) // 7
    end = jnp.arange(49, dtype=jnp.int32) % 7
    tbl = jnp.where(
        (d[None, :] >= off[:, None]) & (d[None, :] < end[:, None]),
        jnp.float32(0.0), jnp.float32(-jnp.inf),
    )
    tbl = jnp.broadcast_to(tbl[:, :, None], (49, WIN, C))
    out = pl.pallas_call(
        _roi_pool_body,
        grid=(N // K,),
        in_specs=[
            pl.BlockSpec((K, 64), lambda i: (i, 0), memory_space=pltpu.SMEM),
            pl.BlockSpec(memory_space=pltpu.MemorySpace.HBM),
            pl.BlockSpec((49, WIN, C), lambda i: (0, 0, 0)),
        ],
        out_specs=pl.BlockSpec((K, OUT_H, OUT_W, C), lambda i: (i, 0, 0, 0)),
        out_shape=jax.ShapeDtypeStruct((N, OUT_H, OUT_W, C), jnp.float32),
        scratch_shapes=[
            pltpu.VMEM((3, 32, 33, 32, C), jnp.float32),
            pltpu.VMEM((K, OUT_W, 32, C), jnp.float32),
            pltpu.SemaphoreType.DMA,
        ],
        compiler_params=pltpu.CompilerParams(
            dimension_semantics=("arbitrary",),
        ),
    )(params, fT, tbl)
    return jnp.transpose(out, (0, 3, 1, 2))
